# baseline (speedup 1.0000x reference)
import functools

import jax
import jax.numpy as jnp
from jax import lax
from jax.experimental import pallas as pl
from jax.experimental.pallas import tpu as pltpu
from jax.experimental.pallas import tpu_sc as plsc

NUM_EMBEDDINGS = 1000000
EMBEDDING_DIM = 64

_NC = 2
_NS = 16
_NW = _NC * _NS

_CHUNK = 128
_NBUF = 8


def _make_gather(n_total: int):
    assert n_total % (_NW * _CHUNK) == 0
    n_chunks = n_total // (_NW * _CHUNK)
    assert n_chunks % _NBUF == 0
    n_groups = n_chunks // _NBUF

    mesh = plsc.VectorSubcoreMesh(core_axis_name="c", subcore_axis_name="s")

    scratch = [pltpu.VMEM((n_chunks, _CHUNK), jnp.int32)]
    scratch += [pltpu.VMEM((_CHUNK, EMBEDDING_DIM), jnp.float32)
                for _ in range(_NBUF)]
    scratch += [pltpu.SemaphoreType.DMA for _ in range(2 * _NBUF)]

    @functools.partial(
        pl.kernel,
        out_type=jax.ShapeDtypeStruct(
            (n_total // _CHUNK, _CHUNK, EMBEDDING_DIM), jnp.float32),
        mesh=mesh,
        scratch_types=scratch,
        compiler_params=pltpu.CompilerParams(use_tc_tiling_on_sc=False),
    )
    def gather_kernel(ids_hbm, table_hbm, out_hbm, idx_v, *bufs_and_sems):
        bufs = bufs_and_sems[:_NBUF]
        gsems = bufs_and_sems[_NBUF:2 * _NBUF]
        osems = bufs_and_sems[2 * _NBUF:]

        wid = lax.axis_index("s") * _NC + lax.axis_index("c")
        chunk0 = wid * n_chunks

        pltpu.sync_copy(ids_hbm.at[pl.ds(chunk0, n_chunks)], idx_v)

        def start_gather(c, b):
            pltpu.make_async_copy(
                table_hbm.at[idx_v.at[c]], bufs[b], gsems[b]).start()

        def wait_gather(c, b):
            pltpu.make_async_copy(
                table_hbm.at[idx_v.at[c]], bufs[b], gsems[b]).wait()

        def start_store(c, b):
            pltpu.make_async_copy(
                bufs[b], out_hbm.at[chunk0 + c], osems[b]).start()

        def wait_store(c, b):
            pltpu.make_async_copy(
                bufs[b], out_hbm.at[chunk0 + c], osems[b]).wait()

        for b in range(_NBUF):
            start_gather(b, b)

        def body(g, carry):
            for b in range(_NBUF):
                c = g * _NBUF + b
                wait_gather(c, b)
                start_store(c, b)
            for b in range(_NBUF):
                c = g * _NBUF + b
                wait_store(c, b)
                start_gather(c + _NBUF, b)
            return carry

        lax.fori_loop(0, n_groups - 1, body, 0, unroll=False)

        g = n_groups - 1
        for b in range(_NBUF):
            c = g * _NBUF + b
            wait_gather(c, b)
            start_store(c, b)
        for b in range(_NBUF):
            wait_store(g * _NBUF + b, b)

    return gather_kernel


def kernel(ids, table):
    batch, n_fields = ids.shape
    n_total = batch * n_fields
    ids_flat = ids.reshape(n_total // _CHUNK, _CHUNK).astype(jnp.int32)
    out = _make_gather(n_total)(ids_flat, table)
    return out.reshape(batch, n_fields, EMBEDDING_DIM)
